# Initial kernel scaffold; baseline (speedup 1.0000x reference)
import jax
import jax.numpy as jnp
from jax.experimental import pallas as pl


def kernel(in_feat, edge_index, W1, b1, W2, b2, Wl, bl):
    raise NotImplementedError("write your pallas kernel here")


# speedup vs baseline: 10.8203x; 10.8203x over previous
import jax
import jax.numpy as jnp
from jax import lax
from jax.experimental import pallas as pl
from jax.experimental.pallas import tpu as pltpu
from jax.experimental.pallas import tpu_sc as plsc

N = 10000
E = 320000
F = 128
H = 8
NCLS = 40

NC = 2
NS = 16
NW = NC * NS

ED = E // NW
ACC_N = 10240
RB = ACC_N // NS

CHUNK = 128
EPT = 10240
NCH = EPT // CHUNK
EPAD = NW * EPT
JUNK = ACC_N - 1

_MESH = plsc.VectorSubcoreMesh(core_axis_name="c", subcore_axis_name="s")
_SC_PARAMS = pltpu.CompilerParams(needs_layout_passes=False,
                                  use_tc_tiling_on_sc=False)


def _deg_body(src_hbm, dst_hbm, out_hbm, sidx, didx, acc_s, acc_d, sem):
    c = lax.axis_index("c")
    s = lax.axis_index("s")
    w = c * NS + s
    cp = pltpu.async_copy(src_hbm.at[w], sidx, sem)
    cp2 = pltpu.async_copy(dst_hbm.at[w], didx, sem)

    zeros = jnp.zeros((16,), jnp.float32)

    def zbody(i, carry):
        acc_s[pl.ds(i * 16, 16)] = zeros
        acc_d[pl.ds(i * 16, 16)] = zeros
        return carry

    lax.fori_loop(0, ACC_N // 16, zbody, 0)
    cp.wait()
    cp2.wait()

    ones = jnp.ones((16,), jnp.float32)

    def ebody(i, carry):
        sv = sidx[pl.ds(i * 16, 16)]
        dv = didx[pl.ds(i * 16, 16)]
        plsc.addupdate_scatter(acc_s, [sv], ones)
        plsc.addupdate_scatter(acc_d, [dv], ones)
        return carry

    lax.fori_loop(0, ED // 16, ebody, 0)
    pltpu.sync_copy(acc_s, out_hbm.at[w, 0])
    pltpu.sync_copy(acc_d, out_hbm.at[w, 1])


_deg_call = pl.kernel(
    _deg_body,
    out_type=jax.ShapeDtypeStruct((NW, 2, ACC_N), jnp.float32),
    mesh=_MESH,
    scratch_types=[
        pltpu.VMEM((ED,), jnp.int32),
        pltpu.VMEM((ED,), jnp.int32),
        pltpu.VMEM((ACC_N,), jnp.float32),
        pltpu.VMEM((ACC_N,), jnp.float32),
        pltpu.SemaphoreType.DMA,
    ],
    compiler_params=_SC_PARAMS,
)


def _msg_body(h_hbm, src_hbm, dst_hbm, z_hbm, out_hbm, sidx, didx, rows, acc, sem):
    c = lax.axis_index("c")
    s = lax.axis_index("s")
    w = c * NS + s
    cp = pltpu.async_copy(src_hbm.at[w], sidx, sem)
    cp2 = pltpu.async_copy(dst_hbm.at[w], didx, sem)
    pltpu.sync_copy(z_hbm.at[pl.ds(s * RB, RB)], acc.at[pl.ds(s * RB, RB)])
    cp.wait()
    cp2.wait()
    plsc.subcore_barrier()

    def ebody(g, carry):
        pltpu.async_copy(h_hbm.at[sidx.at[g]], rows, sem).wait()
        pltpu.sync_copy(rows, acc.at[didx.at[g]], add=True)
        return carry

    lax.fori_loop(0, NCH, ebody, 0)
    plsc.subcore_barrier()
    pltpu.sync_copy(acc.at[pl.ds(s * RB, RB)], out_hbm.at[c, pl.ds(s * RB, RB)])


_msg_call = pl.kernel(
    _msg_body,
    out_type=jax.ShapeDtypeStruct((NC, ACC_N, H), jnp.float32),
    mesh=_MESH,
    scratch_types=[
        pltpu.VMEM((NCH, CHUNK), jnp.int32),
        pltpu.VMEM((NCH, CHUNK), jnp.int32),
        pltpu.VMEM((CHUNK, H), jnp.float32),
        pltpu.VMEM_SHARED((ACC_N, H), jnp.float32),
        pltpu.SemaphoreType.DMA,
    ],
    compiler_params=_SC_PARAMS,
)


def _norm_mm_body(degT_ref, x_ref, w1_ref, h_ref, norms_ref):
    deg = jnp.sum(degT_ref[...], axis=2)
    norms = lax.rsqrt(jnp.maximum(deg, 1.0))
    norms_ref[...] = norms
    nsrc = norms[:N, 0:1]
    xs = x_ref[...] * nsrc
    h_ref[...] = jnp.dot(xs, w1_ref[...], preferred_element_type=jnp.float32)


def _mid_body(agg_ref, norms_ref, b1_ref, w2_ref, out_ref):
    a = agg_ref[0, :N, :] + agg_ref[1, :N, :]
    ndst = norms_ref[:N, 1:2]
    nsrc = norms_ref[:N, 0:1]
    t = jnp.maximum(a * ndst + b1_ref[...], 0.0)
    out_ref[...] = jnp.dot(t * nsrc, w2_ref[...],
                           preferred_element_type=jnp.float32)


def _final_body(agg_ref, norms_ref, b2_ref, wl_ref, bl_ref, out_ref):
    a = agg_ref[0, :N, :] + agg_ref[1, :N, :]
    ndst = norms_ref[:N, 1:2]
    t = jnp.maximum(a * ndst + b2_ref[...], 0.0)
    y = jnp.dot(t, wl_ref[...], preferred_element_type=jnp.float32)
    out_ref[...] = jnp.maximum(y + bl_ref[...], 0.0)


def kernel(in_feat, edge_index, W1, b1, W2, b2, Wl, bl):
    src = edge_index[0].astype(jnp.int32)
    dst = edge_index[1].astype(jnp.int32)

    degp = _deg_call(src.reshape(NW, ED), dst.reshape(NW, ED))
    degpT = degp.transpose(2, 1, 0)

    h1, norms = pl.pallas_call(
        _norm_mm_body,
        out_shape=(
            jax.ShapeDtypeStruct((N, H), jnp.float32),
            jax.ShapeDtypeStruct((ACC_N, 2), jnp.float32),
        ),
    )(degpT, in_feat, W1)

    pad_src = jnp.zeros((EPAD - E,), jnp.int32)
    pad_dst = jnp.full((EPAD - E,), JUNK, jnp.int32)
    src3 = jnp.concatenate([src, pad_src]).reshape(NW, NCH, CHUNK)
    dst3 = jnp.concatenate([dst, pad_dst]).reshape(NW, NCH, CHUNK)
    zrows = jnp.zeros((ACC_N, H), jnp.float32)

    agg1 = _msg_call(h1, src3, dst3, zrows)

    h2 = pl.pallas_call(
        _mid_body,
        out_shape=jax.ShapeDtypeStruct((N, H), jnp.float32),
    )(agg1, norms, b1.reshape(1, H), W2)

    agg2 = _msg_call(h2, src3, dst3, zrows)

    out = pl.pallas_call(
        _final_body,
        out_shape=jax.ShapeDtypeStruct((N, NCLS), jnp.float32),
    )(agg2, norms, b2.reshape(1, H), Wl, bl.reshape(1, NCLS))
    return out


# speedup vs baseline: 14.2102x; 1.3133x over previous
import jax
import jax.numpy as jnp
from jax import lax
from jax.experimental import pallas as pl
from jax.experimental.pallas import tpu as pltpu
from jax.experimental.pallas import tpu_sc as plsc

N = 10000
E = 320000
F = 128
H = 8
NCLS = 40

NC = 2
NS = 16
NW = NC * NS

ED = E // NW
ACC_N = 10240
RB = ACC_N // NS

CHUNK = 128
EPT = 10240
NCH = EPT // CHUNK
EPAD = NW * EPT
JUNK = ACC_N - 1

_MESH = plsc.VectorSubcoreMesh(core_axis_name="c", subcore_axis_name="s")
_SC_PARAMS = pltpu.CompilerParams(needs_layout_passes=False,
                                  use_tc_tiling_on_sc=False)


def _deg_body(src_hbm, dst_hbm, out_hbm, sidx, didx, acc_s, acc_d, sem):
    c = lax.axis_index("c")
    s = lax.axis_index("s")
    w = c * NS + s
    cp = pltpu.async_copy(src_hbm.at[w], sidx, sem)
    cp2 = pltpu.async_copy(dst_hbm.at[w], didx, sem)

    zeros = jnp.zeros((16,), jnp.float32)

    def zbody(i, carry):
        acc_s[pl.ds(i * 16, 16)] = zeros
        acc_d[pl.ds(i * 16, 16)] = zeros
        return carry

    lax.fori_loop(0, ACC_N // 16, zbody, 0)
    cp.wait()
    cp2.wait()

    ones = jnp.ones((16,), jnp.float32)

    def ebody(i, carry):
        sv = sidx[pl.ds(i * 16, 16)]
        dv = didx[pl.ds(i * 16, 16)]
        plsc.addupdate_scatter(acc_s, [sv], ones)
        plsc.addupdate_scatter(acc_d, [dv], ones)
        return carry

    lax.fori_loop(0, ED // 16, ebody, 0)
    pltpu.sync_copy(acc_s, out_hbm.at[w, 0])
    pltpu.sync_copy(acc_d, out_hbm.at[w, 1])


_deg_call = pl.kernel(
    _deg_body,
    out_type=jax.ShapeDtypeStruct((NW, 2, ACC_N), jnp.float32),
    mesh=_MESH,
    scratch_types=[
        pltpu.VMEM((ED,), jnp.int32),
        pltpu.VMEM((ED,), jnp.int32),
        pltpu.VMEM((ACC_N,), jnp.float32),
        pltpu.VMEM((ACC_N,), jnp.float32),
        pltpu.SemaphoreType.DMA,
    ],
    compiler_params=_SC_PARAMS,
)


RING = 8
PREF = 4


def _msg_body(h_hbm, src_hbm, dst_hbm, z_hbm, out_hbm, sidx, didx,
              r0, r1, r2, r3, r4, r5, r6, r7, acc, sem, gsem, ssem):
    rows = (r0, r1, r2, r3, r4, r5, r6, r7)
    c = lax.axis_index("c")
    s = lax.axis_index("s")
    w = c * NS + s
    cp = pltpu.async_copy(src_hbm.at[w], sidx, sem)
    cp2 = pltpu.async_copy(dst_hbm.at[w], didx, sem)
    pltpu.sync_copy(z_hbm.at[pl.ds(s * RB, RB)], acc.at[pl.ds(s * RB, RB)])
    cp.wait()
    cp2.wait()
    plsc.subcore_barrier()

    for b in range(PREF):
        pltpu.async_copy(h_hbm.at[sidx.at[b]], rows[b], gsem.at[b])

    def obody(o, carry):
        for b in range(RING):
            i = o * RING + b
            pltpu.make_async_copy(h_hbm.at[sidx.at[i]], rows[b],
                                  gsem.at[b]).wait()
            pltpu.async_copy(rows[b], acc.at[didx.at[i]], ssem.at[b],
                             add=True)
            p = (b + PREF) % RING

            @pl.when(jnp.logical_and(i >= PREF, i < NCH - PREF))
            def _():
                pltpu.make_async_copy(rows[p], acc.at[didx.at[i]],
                                      ssem.at[p]).wait()

            @pl.when(i < NCH - PREF)
            def _():
                pltpu.async_copy(h_hbm.at[sidx.at[i + PREF]], rows[p],
                                 gsem.at[p])
        return carry

    lax.fori_loop(0, NCH // RING, obody, 0)
    for b in range(RING):
        pltpu.make_async_copy(rows[b], acc.at[didx.at[0]], ssem.at[b]).wait()
    plsc.subcore_barrier()
    pltpu.sync_copy(acc.at[pl.ds(s * RB, RB)], out_hbm.at[c, pl.ds(s * RB, RB)])


_msg_call = pl.kernel(
    _msg_body,
    out_type=jax.ShapeDtypeStruct((NC, ACC_N, H), jnp.float32),
    mesh=_MESH,
    scratch_types=[
        pltpu.VMEM((NCH, CHUNK), jnp.int32),
        pltpu.VMEM((NCH, CHUNK), jnp.int32),
    ] + [pltpu.VMEM((CHUNK, H), jnp.float32)] * RING + [
        pltpu.VMEM_SHARED((ACC_N, H), jnp.float32),
        pltpu.SemaphoreType.DMA,
        pltpu.SemaphoreType.DMA((RING,)),
        pltpu.SemaphoreType.DMA((RING,)),
    ],
    compiler_params=_SC_PARAMS,
)


def _norm_mm_body(degT_ref, x_ref, w1_ref, h_ref, norms_ref):
    deg = jnp.sum(degT_ref[...], axis=2)
    norms = lax.rsqrt(jnp.maximum(deg, 1.0))
    norms_ref[...] = norms
    nsrc = norms[:N, 0:1]
    xs = x_ref[...] * nsrc
    h_ref[...] = jnp.dot(xs, w1_ref[...], preferred_element_type=jnp.float32)


def _mid_body(agg_ref, norms_ref, b1_ref, w2_ref, out_ref):
    a = agg_ref[0, :N, :] + agg_ref[1, :N, :]
    ndst = norms_ref[:N, 1:2]
    nsrc = norms_ref[:N, 0:1]
    t = jnp.maximum(a * ndst + b1_ref[...], 0.0)
    out_ref[...] = jnp.dot(t * nsrc, w2_ref[...],
                           preferred_element_type=jnp.float32)


def _final_body(agg_ref, norms_ref, b2_ref, wl_ref, bl_ref, out_ref):
    a = agg_ref[0, :N, :] + agg_ref[1, :N, :]
    ndst = norms_ref[:N, 1:2]
    t = jnp.maximum(a * ndst + b2_ref[...], 0.0)
    y = jnp.dot(t, wl_ref[...], preferred_element_type=jnp.float32)
    out_ref[...] = jnp.maximum(y + bl_ref[...], 0.0)


def kernel(in_feat, edge_index, W1, b1, W2, b2, Wl, bl):
    src = edge_index[0].astype(jnp.int32)
    dst = edge_index[1].astype(jnp.int32)

    degp = _deg_call(src.reshape(NW, ED), dst.reshape(NW, ED))
    degpT = degp.transpose(2, 1, 0)

    h1, norms = pl.pallas_call(
        _norm_mm_body,
        out_shape=(
            jax.ShapeDtypeStruct((N, H), jnp.float32),
            jax.ShapeDtypeStruct((ACC_N, 2), jnp.float32),
        ),
    )(degpT, in_feat, W1)

    pad_src = jnp.zeros((EPAD - E,), jnp.int32)
    pad_dst = jnp.full((EPAD - E,), JUNK, jnp.int32)
    src3 = jnp.concatenate([src, pad_src]).reshape(NW, NCH, CHUNK)
    dst3 = jnp.concatenate([dst, pad_dst]).reshape(NW, NCH, CHUNK)
    zrows = jnp.zeros((ACC_N, H), jnp.float32)

    agg1 = _msg_call(h1, src3, dst3, zrows)

    h2 = pl.pallas_call(
        _mid_body,
        out_shape=jax.ShapeDtypeStruct((N, H), jnp.float32),
    )(agg1, norms, b1.reshape(1, H), W2)

    agg2 = _msg_call(h2, src3, dst3, zrows)

    out = pl.pallas_call(
        _final_body,
        out_shape=jax.ShapeDtypeStruct((N, NCLS), jnp.float32),
    )(agg2, norms, b2.reshape(1, H), Wl, bl.reshape(1, NCLS))
    return out


# speedup vs baseline: 18.5342x; 1.3043x over previous
import jax
import jax.numpy as jnp
from jax import lax
from jax.experimental import pallas as pl
from jax.experimental.pallas import tpu as pltpu
from jax.experimental.pallas import tpu_sc as plsc

N = 10000
E = 320000
F = 128
H = 8
NCLS = 40

NC = 2
NS = 16
NW = NC * NS

ED = E // NW
ACC_N = 10240
RB = ACC_N // NS

CHUNK = 128
EPT = 10240
NCH = EPT // CHUNK
EPAD = NW * EPT
JUNK = ACC_N - 1

_MESH = plsc.VectorSubcoreMesh(core_axis_name="c", subcore_axis_name="s")
_SC_PARAMS = pltpu.CompilerParams(needs_layout_passes=False,
                                  use_tc_tiling_on_sc=False)


def _deg_body(src_hbm, dst_hbm, out_hbm, sidx, didx, acc_s, acc_d, sem):
    c = lax.axis_index("c")
    s = lax.axis_index("s")
    w = c * NS + s
    cp = pltpu.async_copy(src_hbm.at[w], sidx, sem)
    cp2 = pltpu.async_copy(dst_hbm.at[w], didx, sem)

    zeros = jnp.zeros((16,), jnp.float32)

    def zbody(i, carry):
        acc_s[pl.ds(i * 16, 16)] = zeros
        acc_d[pl.ds(i * 16, 16)] = zeros
        return carry

    lax.fori_loop(0, ACC_N // 16, zbody, 0)
    cp.wait()
    cp2.wait()

    ones = jnp.ones((16,), jnp.float32)

    def ebody(i, carry):
        sv = sidx[pl.ds(i * 16, 16)]
        dv = didx[pl.ds(i * 16, 16)]
        plsc.addupdate_scatter(acc_s, [sv], ones)
        plsc.addupdate_scatter(acc_d, [dv], ones)
        return carry

    lax.fori_loop(0, ED // 16, ebody, 0)
    pltpu.sync_copy(acc_s, out_hbm.at[w, 0])
    pltpu.sync_copy(acc_d, out_hbm.at[w, 1])


_deg_call = pl.kernel(
    _deg_body,
    out_type=jax.ShapeDtypeStruct((NW, 2, ACC_N), jnp.float32),
    mesh=_MESH,
    scratch_types=[
        pltpu.VMEM((ED,), jnp.int32),
        pltpu.VMEM((ED,), jnp.int32),
        pltpu.VMEM((ACC_N,), jnp.float32),
        pltpu.VMEM((ACC_N,), jnp.float32),
        pltpu.SemaphoreType.DMA,
    ],
    compiler_params=_SC_PARAMS,
)


RING = 8
PREF = 4


def _msg_body(h_hbm, src_hbm, dst_hbm, z_hbm, out_hbm, sidx, didx,
              r0, r1, r2, r3, r4, r5, r6, r7, acc, sem, gsem, ssem):
    rows = (r0, r1, r2, r3, r4, r5, r6, r7)
    c = lax.axis_index("c")
    s = lax.axis_index("s")
    w = c * NS + s
    cp = pltpu.async_copy(src_hbm.at[w], sidx, sem)
    cp2 = pltpu.async_copy(dst_hbm.at[w], didx, sem)
    pltpu.sync_copy(z_hbm.at[pl.ds(s * RB, RB)], acc.at[pl.ds(s * RB, RB)])
    cp.wait()
    cp2.wait()
    plsc.subcore_barrier()

    for b in range(PREF):
        pltpu.async_copy(h_hbm.at[sidx.at[b]], rows[b], gsem.at[b])

    def obody(o, carry):
        for b in range(RING):
            i = o * RING + b
            pltpu.make_async_copy(h_hbm.at[sidx.at[i]], rows[b],
                                  gsem.at[b]).wait()
            pltpu.async_copy(rows[b], acc.at[didx.at[i]], ssem.at[b],
                             add=True)
            p = (b + PREF) % RING

            @pl.when(jnp.logical_and(i >= PREF, i < NCH - PREF))
            def _():
                pltpu.make_async_copy(rows[p], acc.at[didx.at[i]],
                                      ssem.at[p]).wait()

            @pl.when(i < NCH - PREF)
            def _():
                pltpu.async_copy(h_hbm.at[sidx.at[i + PREF]], rows[p],
                                 gsem.at[p])
        return carry

    lax.fori_loop(0, NCH // RING, obody, 0)
    for b in range(RING):
        pltpu.make_async_copy(rows[b], acc.at[didx.at[0]], ssem.at[b]).wait()
    plsc.subcore_barrier()
    pltpu.sync_copy(acc.at[pl.ds(s * RB, RB)], out_hbm.at[c, pl.ds(s * RB, RB)])


_msg_call = pl.kernel(
    _msg_body,
    out_type=jax.ShapeDtypeStruct((NC, ACC_N, H), jnp.float32),
    mesh=_MESH,
    scratch_types=[
        pltpu.VMEM((NCH, CHUNK), jnp.int32),
        pltpu.VMEM((NCH, CHUNK), jnp.int32),
    ] + [pltpu.VMEM((CHUNK, H), jnp.float32)] * RING + [
        pltpu.VMEM_SHARED((ACC_N, H), jnp.float32),
        pltpu.SemaphoreType.DMA,
        pltpu.SemaphoreType.DMA((RING,)),
        pltpu.SemaphoreType.DMA((RING,)),
    ],
    compiler_params=_SC_PARAMS,
)


def _norm_mm_body(degT_ref, x_ref, w1_ref, h_ref, norms_ref):
    deg = jnp.sum(degT_ref[...], axis=2)
    norms = lax.rsqrt(jnp.maximum(deg, 1.0))
    norms_ref[...] = norms
    nsrc = norms[:N, 0:1]
    xs = x_ref[...] * nsrc
    h_ref[...] = jnp.dot(xs, w1_ref[...], preferred_element_type=jnp.float32)


def _mid_body(agg_ref, norms_ref, b1_ref, w2_ref, out_ref):
    a = agg_ref[0, :N, :] + agg_ref[1, :N, :]
    ndst = norms_ref[:N, 1:2]
    nsrc = norms_ref[:N, 0:1]
    t = jnp.maximum(a * ndst + b1_ref[...], 0.0)
    out_ref[...] = jnp.dot(t * nsrc, w2_ref[...],
                           preferred_element_type=jnp.float32)


def _final_body(agg_ref, norms_ref, b2_ref, wl_ref, bl_ref, out_ref):
    a = agg_ref[0, :N, :] + agg_ref[1, :N, :]
    ndst = norms_ref[:N, 1:2]
    t = jnp.maximum(a * ndst + b2_ref[...], 0.0)
    y = jnp.dot(t, wl_ref[...], preferred_element_type=jnp.float32)
    out_ref[...] = jnp.maximum(y + bl_ref[...], 0.0)


def kernel(in_feat, edge_index, W1, b1, W2, b2, Wl, bl):
    src = edge_index[0].astype(jnp.int32)
    dst = edge_index[1].astype(jnp.int32)

    degp = _deg_call(src.reshape(NW, ED), dst.reshape(NW, ED))
    degpT = degp.transpose(2, 1, 0)

    h1, norms = pl.pallas_call(
        _norm_mm_body,
        out_shape=(
            jax.ShapeDtypeStruct((N, H), jnp.float32),
            jax.ShapeDtypeStruct((ACC_N, 2), jnp.float32),
        ),
    )(degpT, in_feat, W1)

    npad = EPT - ED
    pad_s = jnp.broadcast_to(jnp.arange(npad, dtype=jnp.int32)[None],
                             (NW, npad))
    pad_d = jnp.broadcast_to((N + jnp.arange(npad, dtype=jnp.int32))[None],
                             (NW, npad))
    src3 = jnp.concatenate([src.reshape(NW, ED), pad_s],
                           axis=1).reshape(NW, NCH, CHUNK)
    dst3 = jnp.concatenate([dst.reshape(NW, ED), pad_d],
                           axis=1).reshape(NW, NCH, CHUNK)
    zrows = jnp.zeros((ACC_N, H), jnp.float32)

    agg1 = _msg_call(h1, src3, dst3, zrows)

    h2 = pl.pallas_call(
        _mid_body,
        out_shape=jax.ShapeDtypeStruct((N, H), jnp.float32),
    )(agg1, norms, b1.reshape(1, H), W2)

    agg2 = _msg_call(h2, src3, dst3, zrows)

    out = pl.pallas_call(
        _final_body,
        out_shape=jax.ShapeDtypeStruct((N, NCLS), jnp.float32),
    )(agg2, norms, b2.reshape(1, H), Wl, bl.reshape(1, NCLS))
    return out


# speedup vs baseline: 31.3200x; 1.6898x over previous
import jax
import jax.numpy as jnp
from jax import lax
from jax.experimental import pallas as pl
from jax.experimental.pallas import tpu as pltpu
from jax.experimental.pallas import tpu_sc as plsc

N = 10000
E = 320000
F = 128
H = 8
NCLS = 40

NC = 2
NS = 16
NW = NC * NS

ED = E // NW
ACC_N = 10240
RB = ACC_N // NS

CHUNK = 128
EPT = 10240
NCH = EPT // CHUNK
EPAD = NW * EPT
JUNK = ACC_N - 1

_MESH = plsc.VectorSubcoreMesh(core_axis_name="c", subcore_axis_name="s")
_SC_PARAMS = pltpu.CompilerParams(needs_layout_passes=False,
                                  use_tc_tiling_on_sc=False)


NPK = ACC_N * H // 128
PKT = NPK // NS


def _deg_body(ei_hbm, degpk_hbm, osrc_hbm, odst_hbm,
              idx_s, idx_d, acc_s, acc_d, slab, tsbuf, rep, sem):
    c = lax.axis_index("c")
    s = lax.axis_index("s")
    w = c * NS + s
    cp = pltpu.async_copy(ei_hbm.at[0, pl.ds(w * ED, ED)],
                          idx_s.at[pl.ds(0, ED)], sem)
    cp2 = pltpu.async_copy(ei_hbm.at[1, pl.ds(w * ED, ED)],
                           idx_d.at[pl.ds(0, ED)], sem)

    zeros = jnp.zeros((16,), jnp.float32)

    def zbody(i, carry):
        acc_s[pl.ds(i * 16, 16)] = zeros
        acc_d[pl.ds(i * 16, 16)] = zeros
        return carry

    lax.fori_loop(0, ACC_N // 16, zbody, 0)

    iota = lax.iota(jnp.int32, 16)
    for k in range((EPT - ED) // 16):
        idx_s[pl.ds(ED + 16 * k, 16)] = iota + (16 * k)
        idx_d[pl.ds(ED + 16 * k, 16)] = iota + (N + 16 * k)

    cp.wait()
    cp2.wait()
    cp3 = pltpu.async_copy(idx_s, osrc_hbm.at[w], sem)
    cp4 = pltpu.async_copy(idx_d, odst_hbm.at[w], sem)

    ones = jnp.ones((16,), jnp.float32)

    def ebody(i, carry):
        sv = idx_s[pl.ds(i * 16, 16)]
        dv = idx_d[pl.ds(i * 16, 16)]
        plsc.addupdate_scatter(acc_s, [sv], ones)
        plsc.addupdate_scatter(acc_d, [dv], ones)
        return carry

    lax.fori_loop(0, ED // 16, ebody, 0)
    pltpu.sync_copy(acc_s, slab.at[s, 0])
    pltpu.sync_copy(acc_d, slab.at[s, 1])
    plsc.subcore_barrier()
    for k in range(NS):
        pltpu.sync_copy(slab.at[k, 0, pl.ds(s * RB, RB)], tsbuf.at[k, 0])
        pltpu.sync_copy(slab.at[k, 1, pl.ds(s * RB, RB)], tsbuf.at[k, 1])
    for kind in range(2):
        def rbody(i, carry):
            v = tsbuf[0, kind, pl.ds(i * 16, 16)]
            for k in range(1, NS):
                v = v + tsbuf[k, kind, pl.ds(i * 16, 16)]
            for j in range(H):
                plsc.store_scatter(rep.at[i], [iota * H + j], v)
            return carry

        lax.fori_loop(0, RB // 16, rbody, 0)
        pltpu.sync_copy(rep, degpk_hbm.at[c, kind, pl.ds(s * PKT, PKT)])
    cp3.wait()
    cp4.wait()


_deg_call = pl.kernel(
    _deg_body,
    out_type=(
        jax.ShapeDtypeStruct((NC, 2, NPK, 128), jnp.float32),
        jax.ShapeDtypeStruct((NW, EPT), jnp.int32),
        jax.ShapeDtypeStruct((NW, EPT), jnp.int32),
    ),
    mesh=_MESH,
    scratch_types=[
        pltpu.VMEM((EPT,), jnp.int32),
        pltpu.VMEM((EPT,), jnp.int32),
        pltpu.VMEM((ACC_N,), jnp.float32),
        pltpu.VMEM((ACC_N,), jnp.float32),
        pltpu.VMEM_SHARED((NS, 2, ACC_N), jnp.float32),
        pltpu.VMEM((NS, 2, RB), jnp.float32),
        pltpu.VMEM((PKT, 128), jnp.float32),
        pltpu.SemaphoreType.DMA,
    ],
    compiler_params=_SC_PARAMS,
)


RING = 8
PREF = 4


def _msg_body(h_hbm, src_hbm, dst_hbm, z_hbm, out_hbm, sidx, didx,
              r0, r1, r2, r3, r4, r5, r6, r7, acc, sem, gsem, ssem):
    rows = (r0, r1, r2, r3, r4, r5, r6, r7)
    c = lax.axis_index("c")
    s = lax.axis_index("s")
    w = c * NS + s
    cp = pltpu.async_copy(src_hbm.at[w], sidx, sem)
    cp2 = pltpu.async_copy(dst_hbm.at[w], didx, sem)
    pltpu.sync_copy(z_hbm.at[pl.ds(s * RB, RB)], acc.at[pl.ds(s * RB, RB)])
    cp.wait()
    cp2.wait()
    plsc.subcore_barrier()

    for b in range(PREF):
        pltpu.async_copy(h_hbm.at[sidx.at[b]], rows[b], gsem.at[b])

    def obody(o, carry):
        for b in range(RING):
            i = o * RING + b
            pltpu.make_async_copy(h_hbm.at[sidx.at[i]], rows[b],
                                  gsem.at[b]).wait()
            pltpu.async_copy(rows[b], acc.at[didx.at[i]], ssem.at[b],
                             add=True)
            p = (b + PREF) % RING

            @pl.when(jnp.logical_and(i >= PREF, i < NCH - PREF))
            def _():
                pltpu.make_async_copy(rows[p], acc.at[didx.at[i]],
                                      ssem.at[p]).wait()

            @pl.when(i < NCH - PREF)
            def _():
                pltpu.async_copy(h_hbm.at[sidx.at[i + PREF]], rows[p],
                                 gsem.at[p])
        return carry

    lax.fori_loop(0, NCH // RING, obody, 0)
    for b in range(RING):
        pltpu.make_async_copy(rows[b], acc.at[didx.at[0]], ssem.at[b]).wait()
    plsc.subcore_barrier()
    pltpu.sync_copy(acc.at[pl.ds(s * RB, RB)], out_hbm.at[c, pl.ds(s * RB, RB)])


_msg_call = pl.kernel(
    _msg_body,
    out_type=jax.ShapeDtypeStruct((NC, ACC_N, H), jnp.float32),
    mesh=_MESH,
    scratch_types=[
        pltpu.VMEM((NCH, CHUNK), jnp.int32),
        pltpu.VMEM((NCH, CHUNK), jnp.int32),
    ] + [pltpu.VMEM((CHUNK, H), jnp.float32)] * RING + [
        pltpu.VMEM_SHARED((ACC_N, H), jnp.float32),
        pltpu.SemaphoreType.DMA,
        pltpu.SemaphoreType.DMA((RING,)),
        pltpu.SemaphoreType.DMA((RING,)),
    ],
    compiler_params=_SC_PARAMS,
)


NP = N * H // 128


def _norm_mm_body(degpk_ref, xp_ref, w1bd_ref, h_ref, norms_ref):
    deg = degpk_ref[0] + degpk_ref[1]
    norms = lax.rsqrt(jnp.maximum(deg, 1.0))
    norms_ref[...] = norms
    y = jnp.dot(xp_ref[...], w1bd_ref[...], preferred_element_type=jnp.float32)
    h_ref[...] = y * norms[0, :NP]


def _mid_body(aggp_ref, norms_ref, b1t_ref, w2bd_ref, out_ref):
    a = aggp_ref[0] + aggp_ref[1]
    t = jnp.maximum(a * norms_ref[1] + b1t_ref[...], 0.0)
    t = (t * norms_ref[0])[:NP]
    out_ref[...] = jnp.dot(t, w2bd_ref[...], preferred_element_type=jnp.float32)


def _final_body(aggp_ref, norms_ref, b2t_ref, wlbd_ref, blt_ref, out_ref):
    a = aggp_ref[0] + aggp_ref[1]
    t = jnp.maximum(a * norms_ref[1] + b2t_ref[...], 0.0)
    y = jnp.dot(t[:NP], wlbd_ref[...], preferred_element_type=jnp.float32)
    out_ref[...] = jnp.maximum(y + blt_ref[...], 0.0)


def kernel(in_feat, edge_index, W1, b1, W2, b2, Wl, bl):
    ei = edge_index.astype(jnp.int32)

    degpk, osrc, odst = _deg_call(ei)
    src3 = osrc.reshape(NW, NCH, CHUNK)
    dst3 = odst.reshape(NW, NCH, CHUNK)

    eye16 = jnp.eye(16, dtype=jnp.float32)
    h1, norms = pl.pallas_call(
        _norm_mm_body,
        out_shape=(
            jax.ShapeDtypeStruct((NP, 128), jnp.float32),
            jax.ShapeDtypeStruct((2, NPK, 128), jnp.float32),
        ),
    )(degpk, in_feat.reshape(NP, 16 * F), jnp.kron(eye16, W1))

    zrows = jnp.zeros((ACC_N, H), jnp.float32)
    agg1 = _msg_call(h1.reshape(N, H), src3, dst3, zrows)

    h2 = pl.pallas_call(
        _mid_body,
        out_shape=jax.ShapeDtypeStruct((NP, 128), jnp.float32),
    )(agg1.reshape(NC, NPK, 128), norms, jnp.tile(b1, 16)[None],
      jnp.kron(eye16, W2))

    agg2 = _msg_call(h2.reshape(N, H), src3, dst3, zrows)

    out = pl.pallas_call(
        _final_body,
        out_shape=jax.ShapeDtypeStruct((NP, 16 * NCLS), jnp.float32),
    )(agg2.reshape(NC, NPK, 128), norms, jnp.tile(b2, 16)[None],
      jnp.kron(eye16, Wl), jnp.tile(bl, 16)[None])
    return out.reshape(N, NCLS)


# speedup vs baseline: 33.0278x; 1.0545x over previous
import jax
import jax.numpy as jnp
from jax import lax
from jax.experimental import pallas as pl
from jax.experimental.pallas import tpu as pltpu
from jax.experimental.pallas import tpu_sc as plsc

N = 10000
E = 320000
F = 128
H = 8
NCLS = 40

NC = 2
NS = 16
NW = NC * NS

ED = E // NW
ACC_N = 10240
RB = ACC_N // NS

CHUNK = 128
EPT = 10240
NCH = EPT // CHUNK
EPAD = NW * EPT
JUNK = ACC_N - 1

_MESH = plsc.VectorSubcoreMesh(core_axis_name="c", subcore_axis_name="s")
_SC_PARAMS = pltpu.CompilerParams(needs_layout_passes=False,
                                  use_tc_tiling_on_sc=False)


NPK = ACC_N * H // 128
PKT = NPK // NS


def _deg_body(ei_hbm, degpk_hbm, osrc_hbm, odst_hbm,
              idx_s, idx_d, acc_s, acc_d, slab, tsbuf, rep, sem, sem2):
    c = lax.axis_index("c")
    s = lax.axis_index("s")
    w = c * NS + s
    cp = pltpu.async_copy(ei_hbm.at[0, pl.ds(w * ED, ED)],
                          idx_s.at[pl.ds(0, ED)], sem)
    cp2 = pltpu.async_copy(ei_hbm.at[1, pl.ds(w * ED, ED)],
                           idx_d.at[pl.ds(0, ED)], sem)

    zeros = jnp.zeros((16,), jnp.float32)

    def zbody(i, carry):
        for u in range(4):
            acc_s[pl.ds(i * 64 + u * 16, 16)] = zeros
            acc_d[pl.ds(i * 64 + u * 16, 16)] = zeros
        return carry

    lax.fori_loop(0, ACC_N // 64, zbody, 0)

    iota = lax.iota(jnp.int32, 16)
    for k in range((EPT - ED) // 16):
        idx_s[pl.ds(ED + 16 * k, 16)] = iota + (16 * k)
        idx_d[pl.ds(ED + 16 * k, 16)] = iota + (N + 16 * k)

    cp.wait()
    cp2.wait()
    cp3 = pltpu.async_copy(idx_s, osrc_hbm.at[w], sem2)
    cp4 = pltpu.async_copy(idx_d, odst_hbm.at[w], sem2)

    ones = jnp.ones((16,), jnp.float32)

    def ebody(i, carry):
        for u in range(2):
            sv = idx_s[pl.ds(i * 32 + u * 16, 16)]
            dv = idx_d[pl.ds(i * 32 + u * 16, 16)]
            plsc.addupdate_scatter(acc_s, [sv], ones)
            plsc.addupdate_scatter(acc_d, [dv], ones)
        return carry

    lax.fori_loop(0, ED // 32, ebody, 0)
    plsc.addupdate_scatter(acc_s, [idx_s[pl.ds(ED - 16, 16)]], ones)
    plsc.addupdate_scatter(acc_d, [idx_d[pl.ds(ED - 16, 16)]], ones)
    pltpu.sync_copy(acc_s, slab.at[s, 0])
    pltpu.sync_copy(acc_d, slab.at[s, 1])
    plsc.subcore_barrier()
    fetches = []
    for k in range(NS):
        fetches.append(pltpu.async_copy(
            slab.at[k, 0, pl.ds(s * RB, RB)], tsbuf.at[k, 0], sem))
        fetches.append(pltpu.async_copy(
            slab.at[k, 1, pl.ds(s * RB, RB)], tsbuf.at[k, 1], sem))
    for cp5 in fetches:
        cp5.wait()
    for kind in range(2):
        def rbody(i, carry):
            v = tsbuf[0, kind, pl.ds(i * 16, 16)]
            for k in range(1, NS):
                v = v + tsbuf[k, kind, pl.ds(i * 16, 16)]
            for j in range(H):
                plsc.store_scatter(rep.at[i], [iota * H + j], v)
            return carry

        lax.fori_loop(0, RB // 16, rbody, 0)
        pltpu.sync_copy(rep, degpk_hbm.at[c, kind, pl.ds(s * PKT, PKT)])
    cp3.wait()
    cp4.wait()


_deg_call = pl.kernel(
    _deg_body,
    out_type=(
        jax.ShapeDtypeStruct((NC, 2, NPK, 128), jnp.float32),
        jax.ShapeDtypeStruct((NW, EPT), jnp.int32),
        jax.ShapeDtypeStruct((NW, EPT), jnp.int32),
    ),
    mesh=_MESH,
    scratch_types=[
        pltpu.VMEM((EPT,), jnp.int32),
        pltpu.VMEM((EPT,), jnp.int32),
        pltpu.VMEM((ACC_N,), jnp.float32),
        pltpu.VMEM((ACC_N,), jnp.float32),
        pltpu.VMEM_SHARED((NS, 2, ACC_N), jnp.float32),
        pltpu.VMEM((NS, 2, RB), jnp.float32),
        pltpu.VMEM((PKT, 128), jnp.float32),
        pltpu.SemaphoreType.DMA,
        pltpu.SemaphoreType.DMA,
    ],
    compiler_params=_SC_PARAMS,
)


RING = 8
PREF = 4


def _msg_body(h_hbm, src_hbm, dst_hbm, z_hbm, out_hbm, sidx, didx,
              r0, r1, r2, r3, r4, r5, r6, r7, acc, sem, gsem, ssem):
    rows = (r0, r1, r2, r3, r4, r5, r6, r7)
    c = lax.axis_index("c")
    s = lax.axis_index("s")
    w = c * NS + s
    cp = pltpu.async_copy(src_hbm.at[w], sidx, sem)
    cp2 = pltpu.async_copy(dst_hbm.at[w], didx, sem)
    pltpu.sync_copy(z_hbm.at[pl.ds(s * RB, RB)], acc.at[pl.ds(s * RB, RB)])
    cp.wait()
    cp2.wait()
    plsc.subcore_barrier()

    for b in range(PREF):
        pltpu.async_copy(h_hbm.at[sidx.at[b]], rows[b], gsem.at[b])

    def obody(o, carry):
        for b in range(RING):
            i = o * RING + b
            pltpu.make_async_copy(h_hbm.at[sidx.at[i]], rows[b],
                                  gsem.at[b]).wait()
            pltpu.async_copy(rows[b], acc.at[didx.at[i]], ssem.at[b],
                             add=True)
            p = (b + PREF) % RING

            @pl.when(jnp.logical_and(i >= PREF, i < NCH - PREF))
            def _():
                pltpu.make_async_copy(rows[p], acc.at[didx.at[i]],
                                      ssem.at[p]).wait()

            @pl.when(i < NCH - PREF)
            def _():
                pltpu.async_copy(h_hbm.at[sidx.at[i + PREF]], rows[p],
                                 gsem.at[p])
        return carry

    lax.fori_loop(0, NCH // RING, obody, 0)
    for b in range(RING):
        pltpu.make_async_copy(rows[b], acc.at[didx.at[0]], ssem.at[b]).wait()
    plsc.subcore_barrier()
    pltpu.sync_copy(acc.at[pl.ds(s * RB, RB)], out_hbm.at[c, pl.ds(s * RB, RB)])


_msg_call = pl.kernel(
    _msg_body,
    out_type=jax.ShapeDtypeStruct((NC, ACC_N, H), jnp.float32),
    mesh=_MESH,
    scratch_types=[
        pltpu.VMEM((NCH, CHUNK), jnp.int32),
        pltpu.VMEM((NCH, CHUNK), jnp.int32),
    ] + [pltpu.VMEM((CHUNK, H), jnp.float32)] * RING + [
        pltpu.VMEM_SHARED((ACC_N, H), jnp.float32),
        pltpu.SemaphoreType.DMA,
        pltpu.SemaphoreType.DMA((RING,)),
        pltpu.SemaphoreType.DMA((RING,)),
    ],
    compiler_params=_SC_PARAMS,
)


NP = N * H // 128


def _mm1_body(xp_ref, w1bd_ref, y_ref):
    y_ref[...] = jnp.dot(xp_ref[...], w1bd_ref[...],
                         preferred_element_type=jnp.float32)


def _norm_mm_body(degpk_ref, y_ref, h_ref, norms_ref):
    deg = degpk_ref[0] + degpk_ref[1]
    norms = lax.rsqrt(jnp.maximum(deg, 1.0))
    norms_ref[...] = norms
    h_ref[...] = y_ref[...] * norms[0, :NP]


def _mid_body(aggp_ref, norms_ref, b1t_ref, w2bd_ref, out_ref):
    a = aggp_ref[0] + aggp_ref[1]
    t = jnp.maximum(a * norms_ref[1] + b1t_ref[...], 0.0)
    t = (t * norms_ref[0])[:NP]
    out_ref[...] = jnp.dot(t, w2bd_ref[...], preferred_element_type=jnp.float32)


def _final_body(aggp_ref, norms_ref, b2t_ref, wlbd_ref, blt_ref, out_ref):
    a = aggp_ref[0] + aggp_ref[1]
    t = jnp.maximum(a * norms_ref[1] + b2t_ref[...], 0.0)
    y = jnp.dot(t[:NP], wlbd_ref[...], preferred_element_type=jnp.float32)
    out_ref[...] = jnp.maximum(y + blt_ref[...], 0.0)


def kernel(in_feat, edge_index, W1, b1, W2, b2, Wl, bl):
    ei = edge_index.astype(jnp.int32)

    degpk, osrc, odst = _deg_call(ei)
    src3 = osrc.reshape(NW, NCH, CHUNK)
    dst3 = odst.reshape(NW, NCH, CHUNK)

    eye16 = jnp.eye(16, dtype=jnp.float32)
    y1 = pl.pallas_call(
        _mm1_body,
        out_shape=jax.ShapeDtypeStruct((NP, 128), jnp.float32),
    )(in_feat.reshape(NP, 16 * F), jnp.kron(eye16, W1))

    h1, norms = pl.pallas_call(
        _norm_mm_body,
        out_shape=(
            jax.ShapeDtypeStruct((NP, 128), jnp.float32),
            jax.ShapeDtypeStruct((2, NPK, 128), jnp.float32),
        ),
    )(degpk, y1)

    zrows = jnp.zeros((ACC_N, H), jnp.float32)
    agg1 = _msg_call(h1.reshape(N, H), src3, dst3, zrows)

    h2 = pl.pallas_call(
        _mid_body,
        out_shape=jax.ShapeDtypeStruct((NP, 128), jnp.float32),
    )(agg1.reshape(NC, NPK, 128), norms, jnp.tile(b1, 16)[None],
      jnp.kron(eye16, W2))

    agg2 = _msg_call(h2.reshape(N, H), src3, dst3, zrows)

    out = pl.pallas_call(
        _final_body,
        out_shape=jax.ShapeDtypeStruct((NP, 16 * NCLS), jnp.float32),
    )(agg2.reshape(NC, NPK, 128), norms, jnp.tile(b2, 16)[None],
      jnp.kron(eye16, Wl), jnp.tile(bl, 16)[None])
    return out.reshape(N, NCLS)


# speedup vs baseline: 35.3187x; 1.0694x over previous
import jax
import jax.numpy as jnp
from jax import lax
from jax.experimental import pallas as pl
from jax.experimental.pallas import tpu as pltpu
from jax.experimental.pallas import tpu_sc as plsc

N = 10000
E = 320000
F = 128
H = 8
NCLS = 40

NC = 2
NS = 16
NW = NC * NS

ED = E // NW
ACC_N = 10240
RB = ACC_N // NS

CHUNK = 128
EPT = 10240
NCH = EPT // CHUNK
EPAD = NW * EPT
JUNK = ACC_N - 1

_MESH = plsc.VectorSubcoreMesh(core_axis_name="c", subcore_axis_name="s")
_SC_PARAMS = pltpu.CompilerParams(needs_layout_passes=False,
                                  use_tc_tiling_on_sc=False)


NPK = ACC_N * H // 128
PKT = NPK // NS


def _deg_body(ei_hbm, degpk_hbm, osrc_hbm, odst_hbm,
              idx_s, idx_d, acc_s, acc_d, slab, tsbuf, rep, sem, sem2):
    c = lax.axis_index("c")
    s = lax.axis_index("s")
    w = c * NS + s
    cp = pltpu.async_copy(ei_hbm.at[0, pl.ds(w * ED, ED)],
                          idx_s.at[pl.ds(0, ED)], sem)
    cp2 = pltpu.async_copy(ei_hbm.at[1, pl.ds(w * ED, ED)],
                           idx_d.at[pl.ds(0, ED)], sem)

    zeros = jnp.zeros((16,), jnp.float32)

    def zbody(i, carry):
        for u in range(4):
            acc_s[pl.ds(i * 64 + u * 16, 16)] = zeros
            acc_d[pl.ds(i * 64 + u * 16, 16)] = zeros
        return carry

    lax.fori_loop(0, ACC_N // 64, zbody, 0)

    iota = lax.iota(jnp.int32, 16)
    for k in range((EPT - ED) // 16):
        idx_s[pl.ds(ED + 16 * k, 16)] = iota + (16 * k)
        idx_d[pl.ds(ED + 16 * k, 16)] = iota + (N + 16 * k)

    cp.wait()
    cp2.wait()
    cp3 = pltpu.async_copy(idx_s, osrc_hbm.at[w], sem2)
    cp4 = pltpu.async_copy(idx_d, odst_hbm.at[w], sem2)

    ones = jnp.ones((16,), jnp.float32)

    def ebody(i, carry):
        for u in range(2):
            sv = idx_s[pl.ds(i * 32 + u * 16, 16)]
            dv = idx_d[pl.ds(i * 32 + u * 16, 16)]
            plsc.addupdate_scatter(acc_s, [sv], ones)
            plsc.addupdate_scatter(acc_d, [dv], ones)
        return carry

    lax.fori_loop(0, ED // 32, ebody, 0)
    plsc.addupdate_scatter(acc_s, [idx_s[pl.ds(ED - 16, 16)]], ones)
    plsc.addupdate_scatter(acc_d, [idx_d[pl.ds(ED - 16, 16)]], ones)
    pltpu.sync_copy(acc_s, slab.at[s, 0])
    pltpu.sync_copy(acc_d, slab.at[s, 1])
    plsc.subcore_barrier()
    fetches = []
    for k in range(NS):
        fetches.append(pltpu.async_copy(
            slab.at[k, 0, pl.ds(s * RB, RB)], tsbuf.at[k, 0], sem))
        fetches.append(pltpu.async_copy(
            slab.at[k, 1, pl.ds(s * RB, RB)], tsbuf.at[k, 1], sem))
    for cp5 in fetches:
        cp5.wait()
    for kind in range(2):
        def rbody(i, carry):
            v = tsbuf[0, kind, pl.ds(i * 16, 16)]
            for k in range(1, NS):
                v = v + tsbuf[k, kind, pl.ds(i * 16, 16)]
            for j in range(H):
                plsc.store_scatter(rep.at[i], [iota * H + j], v)
            return carry

        lax.fori_loop(0, RB // 16, rbody, 0)
        pltpu.sync_copy(rep, degpk_hbm.at[c, kind, pl.ds(s * PKT, PKT)])
    cp3.wait()
    cp4.wait()


_deg_call = pl.kernel(
    _deg_body,
    out_type=(
        jax.ShapeDtypeStruct((NC, 2, NPK, 128), jnp.float32),
        jax.ShapeDtypeStruct((NW, EPT), jnp.int32),
        jax.ShapeDtypeStruct((NW, EPT), jnp.int32),
    ),
    mesh=_MESH,
    scratch_types=[
        pltpu.VMEM((EPT,), jnp.int32),
        pltpu.VMEM((EPT,), jnp.int32),
        pltpu.VMEM((ACC_N,), jnp.float32),
        pltpu.VMEM((ACC_N,), jnp.float32),
        pltpu.VMEM_SHARED((NS, 2, ACC_N), jnp.float32),
        pltpu.VMEM((NS, 2, RB), jnp.float32),
        pltpu.VMEM((PKT, 128), jnp.float32),
        pltpu.SemaphoreType.DMA,
        pltpu.SemaphoreType.DMA,
    ],
    compiler_params=_SC_PARAMS,
)


RING = 10
PREF = 5


def _msg_body(h_hbm, src_hbm, dst_hbm, z_hbm, out_hbm, sidx, didx,
              r0, r1, r2, r3, r4, r5, r6, r7, r8, r9, acc, sem, gsem, ssem):
    rows = (r0, r1, r2, r3, r4, r5, r6, r7, r8, r9)
    c = lax.axis_index("c")
    s = lax.axis_index("s")
    w = c * NS + s
    cp = pltpu.async_copy(src_hbm.at[w], sidx, sem)
    cp2 = pltpu.async_copy(dst_hbm.at[w], didx, sem)
    pltpu.sync_copy(z_hbm.at[pl.ds(s * RB, RB)], acc.at[pl.ds(s * RB, RB)])
    cp.wait()
    cp2.wait()
    plsc.subcore_barrier()

    for b in range(PREF):
        pltpu.async_copy(h_hbm.at[sidx.at[b]], rows[b], gsem.at[b])

    def obody(o, carry):
        for b in range(RING):
            i = o * RING + b
            pltpu.make_async_copy(h_hbm.at[sidx.at[i]], rows[b],
                                  gsem.at[b]).wait()
            pltpu.async_copy(rows[b], acc.at[didx.at[i]], ssem.at[b],
                             add=True)
            p = (b + PREF) % RING

            @pl.when(jnp.logical_and(i >= PREF, i < NCH - PREF))
            def _():
                pltpu.make_async_copy(rows[p], acc.at[didx.at[i]],
                                      ssem.at[p]).wait()

            @pl.when(i < NCH - PREF)
            def _():
                pltpu.async_copy(h_hbm.at[sidx.at[i + PREF]], rows[p],
                                 gsem.at[p])
        return carry

    lax.fori_loop(0, NCH // RING, obody, 0)
    for b in range(RING):
        pltpu.make_async_copy(rows[b], acc.at[didx.at[0]], ssem.at[b]).wait()
    plsc.subcore_barrier()
    pltpu.sync_copy(acc.at[pl.ds(s * RB, RB)], out_hbm.at[c, pl.ds(s * RB, RB)])


_msg_call = pl.kernel(
    _msg_body,
    out_type=jax.ShapeDtypeStruct((NC, ACC_N, H), jnp.float32),
    mesh=_MESH,
    scratch_types=[
        pltpu.VMEM((NCH, CHUNK), jnp.int32),
        pltpu.VMEM((NCH, CHUNK), jnp.int32),
    ] + [pltpu.VMEM((CHUNK, H), jnp.float32)] * RING + [
        pltpu.VMEM_SHARED((ACC_N, H), jnp.float32),
        pltpu.SemaphoreType.DMA,
        pltpu.SemaphoreType.DMA((RING,)),
        pltpu.SemaphoreType.DMA((RING,)),
    ],
    compiler_params=_SC_PARAMS,
)


NP = N * H // 128


def _mm1_body(xp_ref, w1bd_ref, y_ref):
    y_ref[...] = jnp.dot(xp_ref[...], w1bd_ref[...],
                         preferred_element_type=jnp.float32)


def _norm_mm_body(degpk_ref, y_ref, h_ref, norms_ref):
    deg = degpk_ref[0] + degpk_ref[1]
    norms = lax.rsqrt(jnp.maximum(deg, 1.0))
    norms_ref[...] = norms
    h_ref[...] = y_ref[...] * norms[0, :NP]


def _mid_body(aggp_ref, norms_ref, b1t_ref, w2bd_ref, out_ref):
    a = aggp_ref[0] + aggp_ref[1]
    t = jnp.maximum(a * norms_ref[1] + b1t_ref[...], 0.0)
    t = (t * norms_ref[0])[:NP]
    out_ref[...] = jnp.dot(t, w2bd_ref[...], preferred_element_type=jnp.float32)


def _final_body(aggp_ref, norms_ref, b2t_ref, wlbd_ref, blt_ref, out_ref):
    a = aggp_ref[0] + aggp_ref[1]
    t = jnp.maximum(a * norms_ref[1] + b2t_ref[...], 0.0)
    y = jnp.dot(t[:NP], wlbd_ref[...], preferred_element_type=jnp.float32)
    out_ref[...] = jnp.maximum(y + blt_ref[...], 0.0)


def kernel(in_feat, edge_index, W1, b1, W2, b2, Wl, bl):
    ei = edge_index.astype(jnp.int32)

    degpk, osrc, odst = _deg_call(ei)
    src3 = osrc.reshape(NW, NCH, CHUNK)
    dst3 = odst.reshape(NW, NCH, CHUNK)

    eye16 = jnp.eye(16, dtype=jnp.float32)
    y1 = pl.pallas_call(
        _mm1_body,
        out_shape=jax.ShapeDtypeStruct((NP, 128), jnp.float32),
    )(in_feat.reshape(NP, 16 * F), jnp.kron(eye16, W1))

    h1, norms = pl.pallas_call(
        _norm_mm_body,
        out_shape=(
            jax.ShapeDtypeStruct((NP, 128), jnp.float32),
            jax.ShapeDtypeStruct((2, NPK, 128), jnp.float32),
        ),
    )(degpk, y1)

    zrows = jnp.zeros((ACC_N, H), jnp.float32)
    agg1 = _msg_call(h1.reshape(N, H), src3, dst3, zrows)

    h2 = pl.pallas_call(
        _mid_body,
        out_shape=jax.ShapeDtypeStruct((NP, 128), jnp.float32),
    )(agg1.reshape(NC, NPK, 128), norms, jnp.tile(b1, 16)[None],
      jnp.kron(eye16, W2))

    agg2 = _msg_call(h2.reshape(N, H), src3, dst3, zrows)

    out = pl.pallas_call(
        _final_body,
        out_shape=jax.ShapeDtypeStruct((NP, 16 * NCLS), jnp.float32),
    )(agg2.reshape(NC, NPK, 128), norms, jnp.tile(b2, 16)[None],
      jnp.kron(eye16, Wl), jnp.tile(bl, 16)[None])
    return out.reshape(N, NCLS)
